# baseline (speedup 1.0000x reference)
import functools

import jax
import jax.numpy as jnp
from jax import lax
from jax.experimental import pallas as pl
from jax.experimental.pallas import tpu as pltpu
from jax.experimental.pallas import tpu_sc as plsc

D_MODEL = 1024
D_HIDDEN = 2048
N_EXP = 8
N_TOK = 2048
BLK = 256
NBLK = 24
PADDED = NBLK * BLK
NC, NS, L = 2, 16, 16
D_GATH = D_MODEL // 2
TPW = N_TOK // NS
SLOTS_PW = PADDED // (NC * NS)
GCH = 24
NBUF = 4
CTOK = N_TOK // (NC * NS)


def _router_body(x_ref, rw_ref, rb_ref, a1_ref, a2_ref, p1_ref, p2_ref):
    logits = lax.dot_general(
        x_ref[...], rw_ref[...], (((1,), (1,)), ((), ())),
        preferred_element_type=jnp.float32) + rb_ref[...]
    iota = lax.broadcasted_iota(jnp.int32, (N_TOK, N_EXP), 1)
    m1 = jnp.max(logits, axis=1, keepdims=True)
    a1 = jnp.min(jnp.where(logits == m1, iota, N_EXP), axis=1, keepdims=True)
    l2 = jnp.where(iota == a1, -jnp.inf, logits)
    m2 = jnp.max(l2, axis=1, keepdims=True)
    a2 = jnp.min(jnp.where(l2 == m2, iota, N_EXP), axis=1, keepdims=True)
    p1 = jax.nn.sigmoid(m1 - m2)
    a1_ref[...] = a1
    a2_ref[...] = a2
    p1_ref[...] = p1
    p2_ref[...] = 1.0 - p1


def _router(x_flat, router_w, router_b):
    return pl.pallas_call(
        _router_body,
        out_shape=[
            jax.ShapeDtypeStruct((N_TOK, 1), jnp.int32),
            jax.ShapeDtypeStruct((N_TOK, 1), jnp.int32),
            jax.ShapeDtypeStruct((N_TOK, 1), jnp.float32),
            jax.ShapeDtypeStruct((N_TOK, 1), jnp.float32),
        ],
    )(x_flat, router_w, router_b.reshape(1, N_EXP))


def _dispatch_body(x_hbm, a1_hbm, a2_hbm, p1_hbm, p2_hbm,
                   xs_hbm, scale_hbm, pos0_hbm, pos1_hbm, be_hbm, nact_hbm,
                   a1b, a2b, p1b, p2b, cnt16, histvm,
                   sidx0, sidx1, tokb, zf, zi,
                   idx48, idx48b, idx48c, idx48d,
                   rows48, rows48b, rows48c, rows48d, svm, bebuf, nactb,
                   tok_sh, scale_sh, hist_sh,
                   dmasem, dmasem2, dmasem3, dmasem4, dmasem5, dmasem6,
                   dmasem7, dmasem8, dmasem9):
    i = lax.axis_index("s")
    cid = lax.axis_index("c")
    lanes = lax.iota(jnp.int32, L)
    zero_i = jnp.zeros((L,), jnp.int32)
    zero_f = jnp.zeros((L,), jnp.float32)

    ld = [pltpu.async_copy(a1_hbm.at[pl.ds(TPW * i, TPW)], a1b, dmasem),
          pltpu.async_copy(a2_hbm.at[pl.ds(TPW * i, TPW)], a2b, dmasem),
          pltpu.async_copy(p1_hbm.at[pl.ds(TPW * i, TPW)], p1b, dmasem),
          pltpu.async_copy(p2_hbm.at[pl.ds(TPW * i, TPW)], p2b, dmasem)]

    nz = PADDED // NS // L
    for c in range(nz):
        zf[pl.ds(L * c, L)] = zero_f
        zi[pl.ds(L * c, L)] = zero_i
    zd = [pltpu.async_copy(
              zi, tok_sh.at[pl.ds((PADDED // NS) * i, PADDED // NS)], dmasem2),
          pltpu.async_copy(
              zf, scale_sh.at[pl.ds((PADDED // NS) * i, PADDED // NS)], dmasem2)]
    for d in ld:
        d.wait()

    sc1 = jax.named_scope("meta_hist")
    sc1.__enter__()
    nj = TPW // L
    idv = ([a1b[pl.ds(L * j, L)] for j in range(nj)] +
           [a2b[pl.ds(L * j, L)] for j in range(nj)])
    one_i = jnp.ones((L,), jnp.int32)
    cntvec = zero_i
    for e in range(N_EXP):
        csp = zero_i
        for v in idv:
            csp = csp + plsc.all_reduce_population_count(v == e)
        cntvec = jnp.where(lanes == e, csp, cntvec)
    cnt16[...] = cntvec
    pltpu.sync_copy(cnt16.at[pl.ds(0, N_EXP)],
                    hist_sh.at[pl.ds(N_EXP * i, N_EXP)])
    for d in zd:
        d.wait()
    sc1.__exit__(None, None, None)
    with jax.named_scope("meta_barrier1"):
        plsc.subcore_barrier()

    sc2 = jax.named_scope("meta_prefix_pos")
    sc2.__enter__()
    pltpu.sync_copy(hist_sh, histvm)
    starts = []
    off_blk = []
    off_slots = zero_i
    nact = zero_i
    for e in range(N_EXP):
        col = plsc.load_gather(histvm, [lanes * N_EXP + e])
        incl = plsc.cumsum(col)
        cnt16[...] = incl
        total = plsc.load_gather(cnt16, [zero_i + (L - 1)])
        myincl = plsc.load_gather(cnt16, [zero_i + i])
        mycol = plsc.load_gather(histvm, [zero_i + (i * N_EXP + e)])
        starts.append(off_slots + myincl - mycol)
        off_blk.append(nact)
        nb = (total + (BLK - 1)) // BLK
        nact = nact + nb
        off_slots = off_slots + nb * BLK

    posv = [zero_i] * (2 * nj)
    for e in range(N_EXP):
        carry = starts[e]
        for j in range(2 * nj):
            m = idv[j] == e
            mi = jnp.where(m, one_i, zero_i)
            c = plsc.cumsum(mi)
            posv[j] = jnp.where(m, carry + c - mi, posv[j])
            carry = carry + plsc.all_reduce_population_count(m)

    for j in range(nj):
        sidx0[pl.ds(L * j, L)] = posv[j]
        sidx1[pl.ds(L * j, L)] = posv[nj + j]
        tokb[pl.ds(L * j, L)] = TPW * i + L * j + lanes

    pd = [pltpu.async_copy(sidx0, pos0_hbm.at[pl.ds(TPW * i, TPW)], dmasem9),
          pltpu.async_copy(sidx1, pos1_hbm.at[pl.ds(TPW * i, TPW)], dmasem9)]

    sd = [pltpu.async_copy(tokb, tok_sh.at[sidx0], dmasem2),
          pltpu.async_copy(tokb, tok_sh.at[sidx1], dmasem2),
          pltpu.async_copy(p1b, scale_sh.at[sidx0], dmasem2),
          pltpu.async_copy(p2b, scale_sh.at[sidx1], dmasem2)]
    for d in sd:
        d.wait()
    sc2.__exit__(None, None, None)
    with jax.named_scope("meta_barrier2"):
        plsc.subcore_barrier()

    spw = PADDED // NS
    pltpu.sync_copy(scale_sh.at[pl.ds(spw * i, spw)], svm)
    sv = pltpu.async_copy(svm, scale_hbm.at[pl.ds(spw * i, spw)], dmasem9)

    @pl.when(i == 0)
    def _():
        for c in range(2):
            bi = lanes + L * c
            val = zero_i
            for e in range(1, N_EXP):
                val = val + jnp.where(bi >= off_blk[e], one_i, zero_i)
            bebuf[pl.ds(L * c, L)] = val
        nactb[...] = nact
        pltpu.sync_copy(bebuf, be_hbm)
        pltpu.sync_copy(nactb, nact_hbm)

    g = cid * NS + i
    nch = SLOTS_PW // GCH
    idxb = [idx48, idx48b, idx48c, idx48d]
    rowsb = [rows48, rows48b, rows48c, rows48d]
    gsem = [dmasem, dmasem2, dmasem3, dmasem4]
    wsem = [dmasem5, dmasem6, dmasem7, dmasem8]
    gd = [None] * nch
    wd = [None] * nch

    def chunk_base(s):
        return SLOTS_PW * g + GCH * s

    sc3 = jax.named_scope("xgather")
    sc3.__enter__()
    for s in range(NBUF):
        pltpu.sync_copy(tok_sh.at[pl.ds(chunk_base(s), GCH)], idxb[s])
        gd[s] = pltpu.async_copy(x_hbm.at[idxb[s]], rowsb[s], gsem[s])
    for s in range(nch):
        b = s % NBUF
        gd[s].wait()
        wd[s] = pltpu.async_copy(
            rowsb[b], xs_hbm.at[pl.ds(chunk_base(s), GCH)], wsem[b])
        if s + NBUF < nch:
            wd[s].wait()
            pltpu.sync_copy(tok_sh.at[pl.ds(chunk_base(s + NBUF), GCH)],
                            idxb[b])
            gd[s + NBUF] = pltpu.async_copy(x_hbm.at[idxb[b]], rowsb[b],
                                            gsem[b])
    sc3.__exit__(None, None, None)
    with jax.named_scope("xgather_drain"):
        for s in range(nch - NBUF, nch):
            wd[s].wait()
        sv.wait()
        for d in pd:
            d.wait()


def _dispatch(x_flat, a1f, a2f, p1f, p2f):
    mesh = plsc.VectorSubcoreMesh(core_axis_name="c", subcore_axis_name="s")
    kern = pl.kernel(
        _dispatch_body,
        out_type=[
            jax.ShapeDtypeStruct((PADDED, D_MODEL), jnp.float32),
            jax.ShapeDtypeStruct((PADDED,), jnp.float32),
            jax.ShapeDtypeStruct((N_TOK,), jnp.int32),
            jax.ShapeDtypeStruct((N_TOK,), jnp.int32),
            jax.ShapeDtypeStruct((2 * L,), jnp.int32),
            jax.ShapeDtypeStruct((L,), jnp.int32),
        ],
        mesh=mesh,
        scratch_types=[
            pltpu.VMEM((TPW,), jnp.int32),
            pltpu.VMEM((TPW,), jnp.int32),
            pltpu.VMEM((TPW,), jnp.float32),
            pltpu.VMEM((TPW,), jnp.float32),
            pltpu.VMEM((L,), jnp.int32),
            pltpu.VMEM((NS * N_EXP,), jnp.int32),
            pltpu.VMEM((TPW,), jnp.int32),
            pltpu.VMEM((TPW,), jnp.int32),
            pltpu.VMEM((TPW,), jnp.int32),
            pltpu.VMEM((PADDED // NS,), jnp.float32),
            pltpu.VMEM((PADDED // NS,), jnp.int32),
            pltpu.VMEM((GCH,), jnp.int32),
            pltpu.VMEM((GCH,), jnp.int32),
            pltpu.VMEM((GCH,), jnp.int32),
            pltpu.VMEM((GCH,), jnp.int32),
            pltpu.VMEM((GCH, D_MODEL), jnp.float32),
            pltpu.VMEM((GCH, D_MODEL), jnp.float32),
            pltpu.VMEM((GCH, D_MODEL), jnp.float32),
            pltpu.VMEM((GCH, D_MODEL), jnp.float32),
            pltpu.VMEM((PADDED // NS,), jnp.float32),
            pltpu.VMEM((2 * L,), jnp.int32),
            pltpu.VMEM((L,), jnp.int32),
            pltpu.VMEM_SHARED((PADDED,), jnp.int32),
            pltpu.VMEM_SHARED((PADDED,), jnp.float32),
            pltpu.VMEM_SHARED((NS * N_EXP,), jnp.int32),
        ] + [pltpu.SemaphoreType.DMA] * 9,
        compiler_params=pltpu.CompilerParams(needs_layout_passes=False),
    )
    return kern(x_flat, a1f, a2f, p1f, p2f)


def _gmm_body(be_ref, nact_ref, xs_ref, scale_ref, w_ref, v_ref, wo_ref,
              y_ref):
    b = pl.program_id(0)

    @pl.when(b < nact_ref[0])
    def _():
        xb = xs_ref[...]
        prec = lax.Precision.DEFAULT
        a = lax.dot_general(xb, w_ref[0], (((1,), (0,)), ((), ())),
                            precision=prec,
                            preferred_element_type=jnp.float32)
        h = lax.dot_general(xb, v_ref[0], (((1,), (0,)), ((), ())),
                            precision=prec,
                            preferred_element_type=jnp.float32)
        hidden = a * (h * jax.nn.sigmoid(h))
        y = lax.dot_general(hidden, wo_ref[0], (((1,), (0,)), ((), ())),
                            precision=prec,
                            preferred_element_type=jnp.float32)
        y_ref[...] = y * scale_ref[...]


def _gmm(be, nact, xs, scale, W, V, W_out):
    grid_spec = pltpu.PrefetchScalarGridSpec(
        num_scalar_prefetch=2,
        grid=(NBLK,),
        in_specs=[
            pl.BlockSpec((BLK, D_MODEL), lambda b, be, na: (b, 0)),
            pl.BlockSpec((BLK, 1), lambda b, be, na: (b, 0)),
            pl.BlockSpec((1, D_MODEL, D_HIDDEN),
                         lambda b, be, na: (be[b], 0, 0)),
            pl.BlockSpec((1, D_MODEL, D_HIDDEN),
                         lambda b, be, na: (be[b], 0, 0)),
            pl.BlockSpec((1, D_HIDDEN, D_MODEL),
                         lambda b, be, na: (be[b], 0, 0)),
        ],
        out_specs=pl.BlockSpec((BLK, D_MODEL), lambda b, be, na: (b, 0)),
    )
    return pl.pallas_call(
        _gmm_body,
        grid_spec=grid_spec,
        out_shape=jax.ShapeDtypeStruct((PADDED, D_MODEL), jnp.float32),
        compiler_params=pltpu.CompilerParams(
            dimension_semantics=("arbitrary",)),
    )(be, nact, xs, scale, W, V, W_out)


def _combine_body(y_hbm, pos0_hbm, pos1_hbm, out_hbm,
                  idxa, idxb, rowsa, rowsb, dmasem):
    i = lax.axis_index("s")
    cid = lax.axis_index("c")
    g = cid * NS + i
    half = CTOK // 2
    for s in range(2):
        tb = CTOK * g + half * s
        pltpu.sync_copy(pos0_hbm.at[pl.ds(tb, half)], idxa)
        pltpu.sync_copy(pos1_hbm.at[pl.ds(tb, half)], idxb)
        pltpu.async_copy(y_hbm.at[idxa], rowsa, dmasem).wait()
        pltpu.async_copy(y_hbm.at[idxb], rowsb, dmasem).wait()

        def rbody(r, _):
            for c in range(D_MODEL // L):
                rowsa[r, pl.ds(L * c, L)] = (
                    rowsa[r, pl.ds(L * c, L)] + rowsb[r, pl.ds(L * c, L)])
            return 0

        lax.fori_loop(0, half, rbody, 0)
        pltpu.sync_copy(rowsa, out_hbm.at[pl.ds(tb, half)])


def _combine(y, pos0, pos1):
    mesh = plsc.VectorSubcoreMesh(core_axis_name="c", subcore_axis_name="s")
    half = CTOK // 2
    kern = pl.kernel(
        _combine_body,
        out_type=jax.ShapeDtypeStruct((N_TOK, D_MODEL), jnp.float32),
        mesh=mesh,
        scratch_types=[
            pltpu.VMEM((half,), jnp.int32),
            pltpu.VMEM((half,), jnp.int32),
            pltpu.VMEM((half, D_MODEL), jnp.float32),
            pltpu.VMEM((half, D_MODEL), jnp.float32),
            pltpu.SemaphoreType.DMA,
        ],
        compiler_params=pltpu.CompilerParams(needs_layout_passes=False),
    )
    return kern(y, pos0, pos1)


def kernel(x, W, V, W_out, router_w, router_b):
    Bb, Tt, D = x.shape
    x_flat = x.reshape(Bb * Tt, D)
    a1, a2, p1, p2 = _router(x_flat, router_w, router_b)
    xs, scale, pos0, pos1, be, nact = _dispatch(
        x_flat, a1.reshape(-1), a2.reshape(-1),
        p1.reshape(-1), p2.reshape(-1))
    y = _gmm(be, nact, xs, scale.reshape(PADDED, 1), W, V, W_out)
    out = _combine(y, pos0, pos1)
    return out.reshape(Bb, Tt, D)


# speedup vs baseline: 1.5105x; 1.5105x over previous
import functools

import jax
import jax.numpy as jnp
from jax import lax
from jax.experimental import pallas as pl
from jax.experimental.pallas import tpu as pltpu
from jax.experimental.pallas import tpu_sc as plsc

D_MODEL = 1024
D_HIDDEN = 2048
N_EXP = 8
N_TOK = 2048
BLK = 256
NBLK = 24
PADDED = NBLK * BLK
NC, NS, L = 2, 16, 16
D_GATH = D_MODEL // 2
TPW = N_TOK // NS
SLOTS_PW = PADDED // (NC * NS)
GCH = 24
NBUF = 4
CTOK = N_TOK // (NC * NS)


def _router_body(x_ref, rw_ref, rb_ref, a1_ref, a2_ref, p1_ref, p2_ref):
    logits = lax.dot_general(
        x_ref[...], rw_ref[...], (((1,), (1,)), ((), ())),
        preferred_element_type=jnp.float32) + rb_ref[...]
    iota = lax.broadcasted_iota(jnp.int32, (N_TOK, N_EXP), 1)
    m1 = jnp.max(logits, axis=1, keepdims=True)
    a1 = jnp.min(jnp.where(logits == m1, iota, N_EXP), axis=1, keepdims=True)
    l2 = jnp.where(iota == a1, -jnp.inf, logits)
    m2 = jnp.max(l2, axis=1, keepdims=True)
    a2 = jnp.min(jnp.where(l2 == m2, iota, N_EXP), axis=1, keepdims=True)
    p1 = jax.nn.sigmoid(m1 - m2)
    a1_ref[...] = a1
    a2_ref[...] = a2
    p1_ref[...] = p1
    p2_ref[...] = 1.0 - p1


def _router(x_flat, router_w, router_b):
    return pl.pallas_call(
        _router_body,
        out_shape=[
            jax.ShapeDtypeStruct((N_TOK, 1), jnp.int32),
            jax.ShapeDtypeStruct((N_TOK, 1), jnp.int32),
            jax.ShapeDtypeStruct((N_TOK, 1), jnp.float32),
            jax.ShapeDtypeStruct((N_TOK, 1), jnp.float32),
        ],
    )(x_flat, router_w, router_b.reshape(1, N_EXP))


def _dispatch_body(a1_hbm, a2_hbm, p1_hbm, p2_hbm,
                   tok_hbm, scale_hbm, pos0_hbm, pos1_hbm, be_hbm, nact_hbm,
                   a1b, a2b, p1b, p2b, cnt16, histvm,
                   sidx0, sidx1, tokb, zf, zi, svm, bebuf, nactb,
                   tok_sh, scale_sh, hist_sh,
                   dmasem, dmasem2, dmasem9):
    i = lax.axis_index("s")
    cid = lax.axis_index("c")
    lanes = lax.iota(jnp.int32, L)
    zero_i = jnp.zeros((L,), jnp.int32)
    zero_f = jnp.zeros((L,), jnp.float32)

    ld = [pltpu.async_copy(a1_hbm.at[pl.ds(TPW * i, TPW)], a1b, dmasem),
          pltpu.async_copy(a2_hbm.at[pl.ds(TPW * i, TPW)], a2b, dmasem),
          pltpu.async_copy(p1_hbm.at[pl.ds(TPW * i, TPW)], p1b, dmasem),
          pltpu.async_copy(p2_hbm.at[pl.ds(TPW * i, TPW)], p2b, dmasem)]

    nz = PADDED // NS // L
    for c in range(nz):
        zf[pl.ds(L * c, L)] = zero_f
        zi[pl.ds(L * c, L)] = zero_i
    zd = [pltpu.async_copy(
              zi, tok_sh.at[pl.ds((PADDED // NS) * i, PADDED // NS)], dmasem2),
          pltpu.async_copy(
              zf, scale_sh.at[pl.ds((PADDED // NS) * i, PADDED // NS)], dmasem2)]
    for d in ld:
        d.wait()

    sc1 = jax.named_scope("meta_hist")
    sc1.__enter__()
    nj = TPW // L
    idv = ([a1b[pl.ds(L * j, L)] for j in range(nj)] +
           [a2b[pl.ds(L * j, L)] for j in range(nj)])
    one_i = jnp.ones((L,), jnp.int32)
    cntvec = zero_i
    for e in range(N_EXP):
        csp = zero_i
        for v in idv:
            csp = csp + plsc.all_reduce_population_count(v == e)
        cntvec = jnp.where(lanes == e, csp, cntvec)
    cnt16[...] = cntvec
    pltpu.sync_copy(cnt16.at[pl.ds(0, N_EXP)],
                    hist_sh.at[pl.ds(N_EXP * i, N_EXP)])
    for d in zd:
        d.wait()
    sc1.__exit__(None, None, None)
    with jax.named_scope("meta_barrier1"):
        plsc.subcore_barrier()

    sc2 = jax.named_scope("meta_prefix_pos")
    sc2.__enter__()
    pltpu.sync_copy(hist_sh, histvm)
    starts = []
    off_blk = []
    off_slots = zero_i
    nact = zero_i
    for e in range(N_EXP):
        col = plsc.load_gather(histvm, [lanes * N_EXP + e])
        incl = plsc.cumsum(col)
        cnt16[...] = incl
        total = plsc.load_gather(cnt16, [zero_i + (L - 1)])
        myincl = plsc.load_gather(cnt16, [zero_i + i])
        mycol = plsc.load_gather(histvm, [zero_i + (i * N_EXP + e)])
        starts.append(off_slots + myincl - mycol)
        off_blk.append(nact)
        nb = (total + (BLK - 1)) // BLK
        nact = nact + nb
        off_slots = off_slots + nb * BLK

    posv = [zero_i] * (2 * nj)
    for e in range(N_EXP):
        carry = starts[e]
        for j in range(2 * nj):
            m = idv[j] == e
            mi = jnp.where(m, one_i, zero_i)
            c = plsc.cumsum(mi)
            posv[j] = jnp.where(m, carry + c - mi, posv[j])
            carry = carry + plsc.all_reduce_population_count(m)

    for j in range(nj):
        sidx0[pl.ds(L * j, L)] = posv[j]
        sidx1[pl.ds(L * j, L)] = posv[nj + j]
        tokb[pl.ds(L * j, L)] = TPW * i + L * j + lanes

    pd = [pltpu.async_copy(sidx0, pos0_hbm.at[pl.ds(TPW * i, TPW)], dmasem9),
          pltpu.async_copy(sidx1, pos1_hbm.at[pl.ds(TPW * i, TPW)], dmasem9)]

    sd = [pltpu.async_copy(tokb, tok_sh.at[sidx0], dmasem2),
          pltpu.async_copy(tokb, tok_sh.at[sidx1], dmasem2),
          pltpu.async_copy(p1b, scale_sh.at[sidx0], dmasem2),
          pltpu.async_copy(p2b, scale_sh.at[sidx1], dmasem2)]
    for d in sd:
        d.wait()
    sc2.__exit__(None, None, None)
    with jax.named_scope("meta_barrier2"):
        plsc.subcore_barrier()

    spw = PADDED // NS
    pltpu.sync_copy(scale_sh.at[pl.ds(spw * i, spw)], svm)
    sv = pltpu.async_copy(svm, scale_hbm.at[pl.ds(spw * i, spw)], dmasem9)
    pltpu.sync_copy(tok_sh.at[pl.ds(spw * i, spw)], zi)
    tv = pltpu.async_copy(zi, tok_hbm.at[pl.ds(spw * i, spw)], dmasem9)

    @pl.when(i == 0)
    def _():
        for c in range(2):
            bi = lanes + L * c
            val = zero_i
            for e in range(1, N_EXP):
                val = val + jnp.where(bi >= off_blk[e], one_i, zero_i)
            bebuf[pl.ds(L * c, L)] = val
        nactb[...] = nact
        pltpu.sync_copy(bebuf, be_hbm)
        pltpu.sync_copy(nactb, nact_hbm)

    with jax.named_scope("meta_drain"):
        sv.wait()
        tv.wait()
        for d in pd:
            d.wait()


def _dispatch(a1f, a2f, p1f, p2f):
    mesh = plsc.VectorSubcoreMesh(core_axis_name="c", subcore_axis_name="s")
    kern = pl.kernel(
        _dispatch_body,
        out_type=[
            jax.ShapeDtypeStruct((PADDED,), jnp.int32),
            jax.ShapeDtypeStruct((PADDED,), jnp.float32),
            jax.ShapeDtypeStruct((N_TOK,), jnp.int32),
            jax.ShapeDtypeStruct((N_TOK,), jnp.int32),
            jax.ShapeDtypeStruct((2 * L,), jnp.int32),
            jax.ShapeDtypeStruct((L,), jnp.int32),
        ],
        mesh=mesh,
        scratch_types=[
            pltpu.VMEM((TPW,), jnp.int32),
            pltpu.VMEM((TPW,), jnp.int32),
            pltpu.VMEM((TPW,), jnp.float32),
            pltpu.VMEM((TPW,), jnp.float32),
            pltpu.VMEM((L,), jnp.int32),
            pltpu.VMEM((NS * N_EXP,), jnp.int32),
            pltpu.VMEM((TPW,), jnp.int32),
            pltpu.VMEM((TPW,), jnp.int32),
            pltpu.VMEM((TPW,), jnp.int32),
            pltpu.VMEM((PADDED // NS,), jnp.float32),
            pltpu.VMEM((PADDED // NS,), jnp.int32),
            pltpu.VMEM((PADDED // NS,), jnp.float32),
            pltpu.VMEM((2 * L,), jnp.int32),
            pltpu.VMEM((L,), jnp.int32),
            pltpu.VMEM_SHARED((PADDED,), jnp.int32),
            pltpu.VMEM_SHARED((PADDED,), jnp.float32),
            pltpu.VMEM_SHARED((NS * N_EXP,), jnp.int32),
        ] + [pltpu.SemaphoreType.DMA] * 3,
        compiler_params=pltpu.CompilerParams(needs_layout_passes=False),
    )
    return kern(a1f, a2f, p1f, p2f)


def _gmm_body(be_ref, nact_ref, tok_ref, scale_ref, x_ref, w_ref, v_ref,
              wo_ref, y_ref):
    b = pl.program_id(0)

    @pl.when(b < nact_ref[0])
    def _():
        tok = tok_ref[0]
        oh = (tok == lax.broadcasted_iota(
            jnp.int32, (BLK, N_TOK), 1)).astype(jnp.float32)
        xb = lax.dot_general(oh, x_ref[...], (((1,), (0,)), ((), ())),
                             preferred_element_type=jnp.float32)
        prec = lax.Precision.DEFAULT
        a = lax.dot_general(xb, w_ref[0], (((1,), (0,)), ((), ())),
                            precision=prec,
                            preferred_element_type=jnp.float32)
        h = lax.dot_general(xb, v_ref[0], (((1,), (0,)), ((), ())),
                            precision=prec,
                            preferred_element_type=jnp.float32)
        hidden = a * (h * jax.nn.sigmoid(h))
        y = lax.dot_general(hidden, wo_ref[0], (((1,), (0,)), ((), ())),
                            precision=prec,
                            preferred_element_type=jnp.float32)
        y_ref[...] = y * scale_ref[...]


def _gmm(be, nact, tok, scale, x_flat, W, V, W_out):
    grid_spec = pltpu.PrefetchScalarGridSpec(
        num_scalar_prefetch=2,
        grid=(NBLK,),
        in_specs=[
            pl.BlockSpec((1, BLK, 1), lambda b, be, na: (b, 0, 0)),
            pl.BlockSpec((BLK, 1), lambda b, be, na: (b, 0)),
            pl.BlockSpec((N_TOK, D_MODEL), lambda b, be, na: (0, 0)),
            pl.BlockSpec((1, D_MODEL, D_HIDDEN),
                         lambda b, be, na: (be[b], 0, 0)),
            pl.BlockSpec((1, D_MODEL, D_HIDDEN),
                         lambda b, be, na: (be[b], 0, 0)),
            pl.BlockSpec((1, D_HIDDEN, D_MODEL),
                         lambda b, be, na: (be[b], 0, 0)),
        ],
        out_specs=pl.BlockSpec((BLK, D_MODEL), lambda b, be, na: (b, 0)),
    )
    return pl.pallas_call(
        _gmm_body,
        grid_spec=grid_spec,
        out_shape=jax.ShapeDtypeStruct((PADDED, D_MODEL), jnp.float32),
        compiler_params=pltpu.CompilerParams(
            dimension_semantics=("arbitrary",),
            vmem_limit_bytes=100 * 1024 * 1024),
    )(be, nact, tok, scale, x_flat, W, V, W_out)


def _combine_body(y_hbm, pos0_hbm, pos1_hbm, out_hbm,
                  idxa, idxb, rowsa, rowsb, dmasem):
    i = lax.axis_index("s")
    cid = lax.axis_index("c")
    g = cid * NS + i
    half = CTOK // 2
    for s in range(2):
        tb = CTOK * g + half * s
        pltpu.sync_copy(pos0_hbm.at[pl.ds(tb, half)], idxa)
        pltpu.sync_copy(pos1_hbm.at[pl.ds(tb, half)], idxb)
        pltpu.async_copy(y_hbm.at[idxa], rowsa, dmasem).wait()
        pltpu.async_copy(y_hbm.at[idxb], rowsb, dmasem).wait()

        def rbody(r, _):
            for c in range(D_MODEL // L):
                rowsa[r, pl.ds(L * c, L)] = (
                    rowsa[r, pl.ds(L * c, L)] + rowsb[r, pl.ds(L * c, L)])
            return 0

        lax.fori_loop(0, half, rbody, 0)
        pltpu.sync_copy(rowsa, out_hbm.at[pl.ds(tb, half)])


def _combine(y, pos0, pos1):
    mesh = plsc.VectorSubcoreMesh(core_axis_name="c", subcore_axis_name="s")
    half = CTOK // 2
    kern = pl.kernel(
        _combine_body,
        out_type=jax.ShapeDtypeStruct((N_TOK, D_MODEL), jnp.float32),
        mesh=mesh,
        scratch_types=[
            pltpu.VMEM((half,), jnp.int32),
            pltpu.VMEM((half,), jnp.int32),
            pltpu.VMEM((half, D_MODEL), jnp.float32),
            pltpu.VMEM((half, D_MODEL), jnp.float32),
            pltpu.SemaphoreType.DMA,
        ],
        compiler_params=pltpu.CompilerParams(needs_layout_passes=False),
    )
    return kern(y, pos0, pos1)


def kernel(x, W, V, W_out, router_w, router_b):
    Bb, Tt, D = x.shape
    x_flat = x.reshape(Bb * Tt, D)
    a1, a2, p1, p2 = _router(x_flat, router_w, router_b)
    tok, scale, pos0, pos1, be, nact = _dispatch(
        a1.reshape(-1), a2.reshape(-1), p1.reshape(-1), p2.reshape(-1))
    y = _gmm(be, nact, tok.reshape(NBLK, BLK, 1), scale.reshape(PADDED, 1),
             x_flat, W, V, W_out)
    out = _combine(y, pos0, pos1)
    return out.reshape(Bb, Tt, D)
